# Initial kernel scaffold; baseline (speedup 1.0000x reference)
import functools

import jax
import jax.numpy as jnp
from jax import lax
from jax.experimental import pallas as pl
from jax.experimental.pallas import tpu as pltpu
from jax.experimental.pallas import tpu_sc as plsc

N = 100000
V = 100000
D_A = 32
D_R = 16
D_N = 16
D_OUT = D_A + D_R + D_N

NW = 32
CB = 448
CPW = 7
LAST = N - CB


def _body(at_hbm, rt_hbm, ex_hbm, wa_hbm, wr_hbm, out_hbm,
          idx_a, idx_r, rows_a, rows_r, ex_v, sem_a, sem_r):
    wid = lax.axis_index("s") * 2 + lax.axis_index("c")
    for c in range(CPW):
        g = wid * CPW + c
        off = jnp.minimum(g * CB, LAST)
        pltpu.sync_copy(at_hbm.at[pl.ds(off, CB)], idx_a)
        pltpu.sync_copy(rt_hbm.at[pl.ds(off, CB)], idx_r)
        ca = pltpu.async_copy(wa_hbm.at[idx_a], rows_a, sem_a)
        cr = pltpu.async_copy(wr_hbm.at[idx_r], rows_r, sem_r)
        pltpu.sync_copy(ex_hbm.at[pl.ds(off, CB), :], ex_v)
        ca.wait()
        cr.wait()
        pltpu.sync_copy(rows_a, out_hbm.at[pl.ds(off, CB), pl.ds(0, D_A)])
        pltpu.sync_copy(rows_r, out_hbm.at[pl.ds(off, CB), pl.ds(D_A, D_R)])
        pltpu.sync_copy(ex_v, out_hbm.at[pl.ds(off, CB), pl.ds(D_A + D_R, D_N)])


@jax.jit
def _run(atom_types, residue_types, extra_feats, W_atom, W_res):
    mesh = plsc.VectorSubcoreMesh(core_axis_name="c", subcore_axis_name="s")
    f = pl.kernel(
        _body,
        mesh=mesh,
        out_type=jax.ShapeDtypeStruct((N, D_OUT), jnp.float32),
        scratch_types=[
            pltpu.VMEM((CB,), jnp.int32),
            pltpu.VMEM((CB,), jnp.int32),
            pltpu.VMEM((CB, D_A), jnp.float32),
            pltpu.VMEM((CB, D_R), jnp.float32),
            pltpu.VMEM((CB, D_N), jnp.float32),
            pltpu.SemaphoreType.DMA,
            pltpu.SemaphoreType.DMA,
        ],
    )
    return f(atom_types, residue_types, extra_feats, W_atom, W_res)


def kernel(atom_types, residue_types, extra_feats, W_atom, W_res):
    return _run(atom_types, residue_types, extra_feats, W_atom, W_res)


# speedup vs baseline: 1.2982x; 1.2982x over previous
import functools

import jax
import jax.numpy as jnp
from jax import lax
from jax.experimental import pallas as pl
from jax.experimental.pallas import tpu as pltpu
from jax.experimental.pallas import tpu_sc as plsc

N = 100000
V = 100000
D_A = 32
D_R = 16
D_N = 16
D_OUT = D_A + D_R + D_N

NW = 32
CB = 448
CPW = 7
LAST = N - CB


def _body(at_hbm, rt_hbm, ex_hbm, wa_hbm, wr_hbm, out_hbm,
          idx_a, idx_r, rows_a, rows_r, ex_v, sem_a, sem_r):
    wid = lax.axis_index("s") * 2 + lax.axis_index("c")
    for c in range(CPW):
        g = wid * CPW + c
        off = jnp.minimum(g * CB, LAST)
        pltpu.sync_copy(at_hbm.at[pl.ds(off, CB)], idx_a)
        pltpu.sync_copy(rt_hbm.at[pl.ds(off, CB)], idx_r)
        ca = pltpu.async_copy(wa_hbm.at[idx_a], rows_a, sem_a)
        cr = pltpu.async_copy(wr_hbm.at[idx_r], rows_r, sem_r)
        pltpu.sync_copy(ex_hbm.at[pl.ds(off, CB), :], ex_v)
        ca.wait()
        cr.wait()
        pltpu.sync_copy(rows_a, out_hbm.at[pl.ds(off, CB), pl.ds(0, D_A)])
        pltpu.sync_copy(rows_r, out_hbm.at[pl.ds(off, CB), pl.ds(D_A, D_R)])
        pltpu.sync_copy(ex_v, out_hbm.at[pl.ds(off, CB), pl.ds(D_A + D_R, D_N)])


@jax.jit
def _run(atom_types, residue_types, extra_feats, W_atom, W_res):
    mesh = plsc.VectorSubcoreMesh(core_axis_name="c", subcore_axis_name="s")
    f = pl.kernel(
        _body,
        mesh=mesh,
        compiler_params=pltpu.CompilerParams(use_tc_tiling_on_sc=False),
        out_type=jax.ShapeDtypeStruct((N, D_OUT), jnp.float32),
        scratch_types=[
            pltpu.VMEM((CB,), jnp.int32),
            pltpu.VMEM((CB,), jnp.int32),
            pltpu.VMEM((CB, D_A), jnp.float32),
            pltpu.VMEM((CB, D_R), jnp.float32),
            pltpu.VMEM((CB, D_N), jnp.float32),
            pltpu.SemaphoreType.DMA,
            pltpu.SemaphoreType.DMA,
        ],
    )
    return f(atom_types, residue_types, extra_feats, W_atom, W_res)


def kernel(atom_types, residue_types, extra_feats, W_atom, W_res):
    return _run(atom_types, residue_types, extra_feats, W_atom, W_res)


# speedup vs baseline: 1.3668x; 1.0528x over previous
import jax
import jax.numpy as jnp
from jax import lax
from jax.experimental import pallas as pl
from jax.experimental.pallas import tpu as pltpu
from jax.experimental.pallas import tpu_sc as plsc

N = 100000
V = 100000
D_A = 32
D_R = 16
D_N = 16
D_OUT = D_A + D_R + D_N

NW = 32
CB = 448
CPW = 7
LAST = N - CB
NBUF = 3


def _body(at_hbm, rt_hbm, ex_hbm, wa_hbm, wr_hbm, out_hbm, *scr):
    idx_a = scr[0:NBUF]
    idx_r = scr[NBUF:2 * NBUF]
    rows_a = scr[2 * NBUF:3 * NBUF]
    rows_r = scr[3 * NBUF:4 * NBUF]
    ex_v = scr[4 * NBUF:5 * NBUF]
    sem_i = scr[5 * NBUF:6 * NBUF]
    sem_g = scr[6 * NBUF:7 * NBUF]
    sem_s = scr[7 * NBUF:8 * NBUF]

    wid = lax.axis_index("s") * 2 + lax.axis_index("c")
    offs = [jnp.minimum((wid * CPW + t) * CB, LAST) for t in range(CPW)]
    d = {}

    def stage_a(t):
        p = t % NBUF
        d["ia", t] = pltpu.async_copy(at_hbm.at[pl.ds(offs[t], CB)], idx_a[p], sem_i[p])
        d["ir", t] = pltpu.async_copy(rt_hbm.at[pl.ds(offs[t], CB)], idx_r[p], sem_i[p])

    def stage_b(t):
        p = t % NBUF
        d["ia", t].wait()
        d["ir", t].wait()
        d["ga", t] = pltpu.async_copy(wa_hbm.at[idx_a[p]], rows_a[p], sem_g[p])
        d["gr", t] = pltpu.async_copy(wr_hbm.at[idx_r[p]], rows_r[p], sem_g[p])
        d["e", t] = pltpu.async_copy(ex_hbm.at[pl.ds(offs[t], CB), :], ex_v[p], sem_g[p])

    def stage_c(t):
        p = t % NBUF
        d["ga", t].wait()
        d["gr", t].wait()
        d["e", t].wait()
        d["sa", t] = pltpu.async_copy(rows_a[p], out_hbm.at[pl.ds(offs[t], CB), pl.ds(0, D_A)], sem_s[p])
        d["sr", t] = pltpu.async_copy(rows_r[p], out_hbm.at[pl.ds(offs[t], CB), pl.ds(D_A, D_R)], sem_s[p])
        d["se", t] = pltpu.async_copy(ex_v[p], out_hbm.at[pl.ds(offs[t], CB), pl.ds(D_A + D_R, D_N)], sem_s[p])

    def drain(t):
        d["sa", t].wait()
        d["sr", t].wait()
        d["se", t].wait()

    for t in range(CPW + 2):
        if t < CPW:
            if t >= NBUF:
                drain(t - NBUF)
            stage_a(t)
        if 1 <= t and t - 1 < CPW:
            stage_b(t - 1)
        if 2 <= t and t - 2 < CPW:
            stage_c(t - 2)
    for t in range(max(0, CPW - NBUF), CPW):
        drain(t)


@jax.jit
def _run(atom_types, residue_types, extra_feats, W_atom, W_res):
    mesh = plsc.VectorSubcoreMesh(core_axis_name="c", subcore_axis_name="s")
    scratch = (
        [pltpu.VMEM((CB,), jnp.int32) for _ in range(NBUF)]
        + [pltpu.VMEM((CB,), jnp.int32) for _ in range(NBUF)]
        + [pltpu.VMEM((CB, D_A), jnp.float32) for _ in range(NBUF)]
        + [pltpu.VMEM((CB, D_R), jnp.float32) for _ in range(NBUF)]
        + [pltpu.VMEM((CB, D_N), jnp.float32) for _ in range(NBUF)]
        + [pltpu.SemaphoreType.DMA for _ in range(3 * NBUF)]
    )
    f = pl.kernel(
        _body,
        mesh=mesh,
        compiler_params=pltpu.CompilerParams(use_tc_tiling_on_sc=False),
        out_type=jax.ShapeDtypeStruct((N, D_OUT), jnp.float32),
        scratch_types=scratch,
    )
    return f(atom_types, residue_types, extra_feats, W_atom, W_res)


def kernel(atom_types, residue_types, extra_feats, W_atom, W_res):
    return _run(atom_types, residue_types, extra_feats, W_atom, W_res)


# speedup vs baseline: 1.5572x; 1.1393x over previous
import jax
import jax.numpy as jnp
from jax import lax
from jax.experimental import pallas as pl
from jax.experimental.pallas import tpu as pltpu
from jax.experimental.pallas import tpu_sc as plsc

N = 100000
V = 100000
D_A = 32
D_R = 16
D_N = 16
D_OUT = D_A + D_R + D_N
W_PAD = 128

NW = 32
CB = 448
CPW = 7
LAST = N - CB
NBUF = 3


def _body(at_hbm, rt_hbm, wa_hbm, wr_hbm, out_hbm, *scr):
    idx_a = scr[0:NBUF]
    idx_r = scr[NBUF:2 * NBUF]
    rows_a = scr[2 * NBUF:3 * NBUF]
    rows_r = scr[3 * NBUF:4 * NBUF]
    sem_i = scr[4 * NBUF:5 * NBUF]
    sem_g = scr[5 * NBUF:6 * NBUF]
    sem_s = scr[6 * NBUF:7 * NBUF]

    wid = lax.axis_index("s") * 2 + lax.axis_index("c")
    offs = [jnp.minimum((wid * CPW + t) * CB, LAST) for t in range(CPW)]
    d = {}

    def stage_a(t):
        p = t % NBUF
        d["ia", t] = pltpu.async_copy(at_hbm.at[pl.ds(offs[t], CB)], idx_a[p], sem_i[p])
        d["ir", t] = pltpu.async_copy(rt_hbm.at[pl.ds(offs[t], CB)], idx_r[p], sem_i[p])

    def stage_b(t):
        p = t % NBUF
        d["ia", t].wait()
        d["ir", t].wait()
        d["ga", t] = pltpu.async_copy(wa_hbm.at[idx_a[p]], rows_a[p], sem_g[p])
        d["gr", t] = pltpu.async_copy(wr_hbm.at[idx_r[p]], rows_r[p], sem_g[p])

    def stage_c(t):
        p = t % NBUF
        d["ga", t].wait()
        d["gr", t].wait()
        d["sa", t] = pltpu.async_copy(rows_a[p], out_hbm.at[pl.ds(offs[t], CB), pl.ds(0, D_A)], sem_s[p])
        d["sr", t] = pltpu.async_copy(rows_r[p], out_hbm.at[pl.ds(offs[t], CB), pl.ds(D_A, D_R)], sem_s[p])

    def drain(t):
        d["sa", t].wait()
        d["sr", t].wait()

    for t in range(CPW + 2):
        if t < CPW:
            if t >= NBUF:
                drain(t - NBUF)
            stage_a(t)
        if 1 <= t and t - 1 < CPW:
            stage_b(t - 1)
        if 2 <= t and t - 2 < CPW:
            stage_c(t - 2)
    for t in range(max(0, CPW - NBUF), CPW):
        drain(t)


@jax.jit
def _run(atom_types, residue_types, extra_feats, W_atom, W_res):
    mesh = plsc.VectorSubcoreMesh(core_axis_name="c", subcore_axis_name="s")
    scratch = (
        [pltpu.VMEM((CB,), jnp.int32) for _ in range(NBUF)]
        + [pltpu.VMEM((CB,), jnp.int32) for _ in range(NBUF)]
        + [pltpu.VMEM((CB, D_A), jnp.float32) for _ in range(NBUF)]
        + [pltpu.VMEM((CB, D_R), jnp.float32) for _ in range(NBUF)]
        + [pltpu.SemaphoreType.DMA for _ in range(3 * NBUF)]
    )
    f = pl.kernel(
        _body,
        mesh=mesh,
        compiler_params=pltpu.CompilerParams(use_tc_tiling_on_sc=False),
        out_type=jax.ShapeDtypeStruct((N, W_PAD), jnp.float32),
        scratch_types=scratch,
    )
    wide = f(atom_types, residue_types, W_atom, W_res)
    return jnp.concatenate([wide[:, :D_A + D_R], extra_feats], axis=1)


def kernel(atom_types, residue_types, extra_feats, W_atom, W_res):
    return _run(atom_types, residue_types, extra_feats, W_atom, W_res)
